# baseline (speedup 1.0000x reference)
import jax
import jax.numpy as jnp
from jax.experimental import pallas as pl


def _body(r_ref, hop_ref, ov_ref):
    v = r_ref[...] * 2.0
    hop_ref[...] = v
    ov_ref[...] = v + 1.0


def kernel(r, bond_indices, edge_numbers, hopping_params, overlap_params,
           atomic_radius_list):
    f32 = jnp.float32
    B = r.shape[0]
    n = B * 13
    rows, cols = n // 1024, 1024
    grid = 64
    blkr = rows // grid
    junk = jnp.broadcast_to(r[:rows].reshape(rows, 1), (rows, cols))
    spec = pl.BlockSpec((blkr, cols), lambda i: (i, 0))
    hop, ov = pl.pallas_call(
        _body,
        grid=(grid,),
        in_specs=[spec],
        out_specs=[spec, spec],
        out_shape=[jax.ShapeDtypeStruct((rows, cols), f32)] * 2,
    )(junk)
    return (hop.reshape(B, 13), ov.reshape(B, 13))


# speedup vs baseline: 1.3192x; 1.3192x over previous
import functools

import jax
import jax.numpy as jnp
from jax import lax
from jax.experimental import pallas as pl
from jax.experimental.pallas import tpu as pltpu
from jax.experimental.pallas import tpu_sc as plsc

_BLK = 2048
_NING = 13


def _r0_body(nw, b_per_w, nc, e0_hbm, e1_hbm, rad_hbm, out_hbm,
             e0_v, e1_v, rad_v, out_v):
    wid = lax.axis_index("s") * nc + lax.axis_index("c")
    base = wid * b_per_w
    pltpu.sync_copy(e0_hbm.at[pl.ds(base, b_per_w)], e0_v)
    pltpu.sync_copy(e1_hbm.at[pl.ds(base, b_per_w)], e1_v)
    pltpu.sync_copy(rad_hbm, rad_v)

    def step(k, _):
        sl = pl.ds(k * 16, 16)
        ia = e0_v[sl]
        ib = e1_v[sl]
        a = plsc.load_gather(rad_v, [ia])
        b = plsc.load_gather(rad_v, [ib])
        out_v[sl] = a + b
        return _

    lax.fori_loop(0, b_per_w // 16, step, None, unroll=4)
    pltpu.sync_copy(out_v, out_hbm.at[pl.ds(base, b_per_w)])


def _sc_r0(e0, e1, rad_pad):
    B = e0.shape[0]
    info = plsc.get_sparse_core_info()
    nc, ns = info.num_cores, info.num_subcores
    nw = nc * ns
    b_per_w = B // nw
    mesh = plsc.VectorSubcoreMesh(core_axis_name="c", subcore_axis_name="s")
    fn = functools.partial(
        pl.kernel,
        mesh=mesh,
        compiler_params=pltpu.CompilerParams(needs_layout_passes=False),
        out_type=jax.ShapeDtypeStruct((B,), jnp.float32),
        scratch_types=[
            pltpu.VMEM((b_per_w,), jnp.int32),
            pltpu.VMEM((b_per_w,), jnp.int32),
            pltpu.VMEM((128,), jnp.float32),
            pltpu.VMEM((b_per_w,), jnp.float32),
        ],
    )(functools.partial(_r0_body, nw, b_per_w, nc))
    return fn(e0, e1, rad_pad)


def _tc_body(r_ref, bt_ref, r0_ref, wt_ref, hop_ref, ov_ref):
    f32 = jnp.float32
    rrow = r_ref[0]
    bt = bt_ref[0]
    r0 = r0_ref[0]
    nbt = wt_ref.shape[1]
    blk = rrow.shape[1]

    iota_t = lax.broadcasted_iota(jnp.int32, (nbt, blk), 0)
    oht = jnp.where(iota_t == bt, 1.0, 0.0).astype(f32)

    g = lax.dot_general(wt_ref[...], oht, (((1,), (0,)), ((), ())),
                        preferred_element_type=f32)

    x = rrow / r0 - 1.0
    x2 = x * x
    lnr = jnp.log(r0 / rrow)
    fcut = 1.0 / (1.0 + jnp.exp((rrow - 5.0) * 5.0))

    for t, out_ref in ((0, hop_ref), (1, ov_ref)):
        base = 64 * t
        g0 = g[base:base + 16]
        g1 = g[base + 16:base + 32]
        g2 = g[base + 32:base + 48]
        g3 = g[base + 48:base + 64]
        poly = g0 + g1 * x + g2 * x2
        a3 = 1.0 + jnp.abs(g3)
        out_t = poly * jnp.exp(a3 * lnr) * fcut
        out = out_t.T
        out_ref[...] = out[:, :_NING]


def kernel(r, bond_indices, edge_numbers, hopping_params, overlap_params,
           atomic_radius_list):
    f32 = jnp.float32
    B = r.shape[0]
    blk = _BLK
    nblk = B // blk
    nbt, ning, np_ = hopping_params.shape

    rad_pad = jnp.zeros((128,), f32).at[:atomic_radius_list.shape[0]].set(
        atomic_radius_list)
    r0 = _sc_r0(edge_numbers[0], edge_numbers[1], rad_pad)

    cols = []
    for tbl in (hopping_params, overlap_params):
        for k in range(np_):
            cols.append(jnp.pad(tbl[:, :, k], ((0, 0), (0, 16 - ning))))
    wt = jnp.concatenate(cols, axis=1).T

    r3 = r.reshape(nblk, 1, blk)
    bt3 = bond_indices.reshape(nblk, 1, blk)
    r03 = r0.reshape(nblk, 1, blk)

    row_spec = pl.BlockSpec((1, 1, blk), lambda i: (i, 0, 0))
    out_spec = pl.BlockSpec((blk, ning), lambda i: (i, 0))
    hop, ov = pl.pallas_call(
        _tc_body,
        grid=(nblk,),
        in_specs=[row_spec, row_spec, row_spec,
                  pl.BlockSpec((128, nbt), lambda i: (0, 0))],
        out_specs=[out_spec, out_spec],
        out_shape=[jax.ShapeDtypeStruct((B, ning), f32),
                   jax.ShapeDtypeStruct((B, ning), f32)],
    )(r3, bt3, r03, wt)
    return (hop, ov)


# speedup vs baseline: 3.7905x; 2.8732x over previous
import functools

import jax
import jax.numpy as jnp
from jax import lax
from jax.experimental import pallas as pl
from jax.experimental.pallas import tpu as pltpu
from jax.experimental.pallas import tpu_sc as plsc

_BLK = 2048
_NING = 13


def _r0_body(nw, b_per_w, nc, e0_hbm, e1_hbm, rad_hbm, out_hbm,
             e0_v, e1_v, rad_v, out_v):
    wid = lax.axis_index("s") * nc + lax.axis_index("c")
    base = wid * b_per_w
    pltpu.sync_copy(e0_hbm.at[pl.ds(base, b_per_w)], e0_v)
    pltpu.sync_copy(e1_hbm.at[pl.ds(base, b_per_w)], e1_v)
    pltpu.sync_copy(rad_hbm, rad_v)

    def step(k, _):
        sl = pl.ds(k * 16, 16)
        ia = e0_v[sl]
        ib = e1_v[sl]
        a = plsc.load_gather(rad_v, [ia])
        b = plsc.load_gather(rad_v, [ib])
        out_v[sl] = a + b
        return _

    lax.fori_loop(0, b_per_w // 16, step, None, unroll=4)
    pltpu.sync_copy(out_v, out_hbm.at[pl.ds(base, b_per_w)])


def _sc_r0(e0, e1, rad_pad):
    B = e0.shape[0]
    info = plsc.get_sparse_core_info()
    nc, ns = info.num_cores, info.num_subcores
    nw = nc * ns
    b_per_w = B // nw
    mesh = plsc.VectorSubcoreMesh(core_axis_name="c", subcore_axis_name="s")
    fn = functools.partial(
        pl.kernel,
        mesh=mesh,
        compiler_params=pltpu.CompilerParams(needs_layout_passes=False),
        out_type=jax.ShapeDtypeStruct((B,), jnp.float32),
        scratch_types=[
            pltpu.VMEM((b_per_w,), jnp.int32),
            pltpu.VMEM((b_per_w,), jnp.int32),
            pltpu.VMEM((128,), jnp.float32),
            pltpu.VMEM((b_per_w,), jnp.float32),
        ],
    )(functools.partial(_r0_body, nw, b_per_w, nc))
    return fn(e0, e1, rad_pad)


def _tc_body(r_ref, bt_ref, r0_ref, wt_ref, hop_ref, ov_ref):
    f32 = jnp.float32
    rrow = r_ref[0]
    bt = bt_ref[0]
    r0 = r0_ref[0]
    nbt = wt_ref.shape[1]
    blk = rrow.shape[1]

    iota_t = lax.broadcasted_iota(jnp.int32, (nbt, blk), 0)
    oht = jnp.where(iota_t == bt, 1.0, 0.0).astype(f32)

    g = lax.dot_general(wt_ref[...], oht, (((1,), (0,)), ((), ())),
                        preferred_element_type=f32)

    x = rrow / r0 - 1.0
    x2 = x * x
    lnr = jnp.log(r0 / rrow)
    fcut = 1.0 / (1.0 + jnp.exp((rrow - 5.0) * 5.0))

    for t, out_ref in ((0, hop_ref), (1, ov_ref)):
        base = 64 * t
        g0 = g[base:base + 16]
        g1 = g[base + 16:base + 32]
        g2 = g[base + 32:base + 48]
        g3 = g[base + 48:base + 64]
        poly = g0 + g1 * x + g2 * x2
        a3 = 1.0 + jnp.abs(g3)
        out_t = poly * jnp.exp(a3 * lnr) * fcut
        out_ref[...] = out_t[:_NING]


def kernel(r, bond_indices, edge_numbers, hopping_params, overlap_params,
           atomic_radius_list):
    f32 = jnp.float32
    B = r.shape[0]
    blk = _BLK
    nblk = B // blk
    nbt, ning, np_ = hopping_params.shape

    rad_pad = jnp.zeros((128,), f32).at[:atomic_radius_list.shape[0]].set(
        atomic_radius_list)
    r0 = _sc_r0(edge_numbers[0], edge_numbers[1], rad_pad)

    cols = []
    for tbl in (hopping_params, overlap_params):
        for k in range(np_):
            cols.append(jnp.pad(tbl[:, :, k], ((0, 0), (0, 16 - ning))))
    wt = jnp.concatenate(cols, axis=1).T

    r3 = r.reshape(nblk, 1, blk)
    bt3 = bond_indices.reshape(nblk, 1, blk)
    r03 = r0.reshape(nblk, 1, blk)

    row_spec = pl.BlockSpec((1, 1, blk), lambda i: (i, 0, 0))
    out_spec = pl.BlockSpec((ning, blk), lambda i: (0, i))
    hop_t, ov_t = pl.pallas_call(
        _tc_body,
        grid=(nblk,),
        in_specs=[row_spec, row_spec, row_spec,
                  pl.BlockSpec((128, nbt), lambda i: (0, 0))],
        out_specs=[out_spec, out_spec],
        out_shape=[jax.ShapeDtypeStruct((ning, B), f32),
                   jax.ShapeDtypeStruct((ning, B), f32)],
    )(r3, bt3, r03, wt)
    return (hop_t.T, ov_t.T)


# speedup vs baseline: 5.5976x; 1.4768x over previous
import functools

import jax
import jax.numpy as jnp
from jax import lax
from jax.experimental import pallas as pl
from jax.experimental.pallas import tpu as pltpu
from jax.experimental.pallas import tpu_sc as plsc

_BLK = 4096
_NING = 13


def _r0_body(nw, b_per_w, nc, e0_hbm, e1_hbm, rad_hbm, out_hbm,
             e0_v, e1_v, rad_v, out_v):
    wid = lax.axis_index("s") * nc + lax.axis_index("c")
    base = wid * b_per_w
    pltpu.sync_copy(e0_hbm.at[pl.ds(base, b_per_w)], e0_v)
    pltpu.sync_copy(e1_hbm.at[pl.ds(base, b_per_w)], e1_v)
    pltpu.sync_copy(rad_hbm, rad_v)

    def step(k, _):
        sl = pl.ds(k * 16, 16)
        ia = e0_v[sl]
        ib = e1_v[sl]
        a = plsc.load_gather(rad_v, [ia])
        b = plsc.load_gather(rad_v, [ib])
        out_v[sl] = a + b
        return _

    lax.fori_loop(0, b_per_w // 16, step, None, unroll=4)
    pltpu.sync_copy(out_v, out_hbm.at[pl.ds(base, b_per_w)])


def _sc_r0(e0, e1, rad_pad):
    B = e0.shape[0]
    info = plsc.get_sparse_core_info()
    nc, ns = info.num_cores, info.num_subcores
    nw = nc * ns
    b_per_w = B // nw
    mesh = plsc.VectorSubcoreMesh(core_axis_name="c", subcore_axis_name="s")
    fn = functools.partial(
        pl.kernel,
        mesh=mesh,
        compiler_params=pltpu.CompilerParams(needs_layout_passes=False),
        out_type=jax.ShapeDtypeStruct((B,), jnp.float32),
        scratch_types=[
            pltpu.VMEM((b_per_w,), jnp.int32),
            pltpu.VMEM((b_per_w,), jnp.int32),
            pltpu.VMEM((128,), jnp.float32),
            pltpu.VMEM((b_per_w,), jnp.float32),
        ],
    )(functools.partial(_r0_body, nw, b_per_w, nc))
    return fn(e0, e1, rad_pad)


def _tc_body(r_ref, bt_ref, r0_ref, wt_ref, hop_ref, ov_ref):
    f32 = jnp.float32
    rrow = r_ref[0]
    bt = bt_ref[0]
    r0 = r0_ref[0]
    nbt = wt_ref.shape[1]
    blk = rrow.shape[1]

    iota_t = lax.broadcasted_iota(jnp.int32, (nbt, blk), 0)
    oht = jnp.where(iota_t == bt, 1.0, 0.0).astype(f32)

    g = lax.dot_general(wt_ref[...], oht, (((1,), (0,)), ((), ())),
                        preferred_element_type=f32)

    x = rrow / r0 - 1.0
    x2 = x * x
    lnr = jnp.log(r0 / rrow)
    fcut = 1.0 / (1.0 + jnp.exp((rrow - 5.0) * 5.0))

    for t, out_ref in ((0, hop_ref), (1, ov_ref)):
        base = 64 * t
        g0 = g[base:base + 16]
        g1 = g[base + 16:base + 32]
        g2 = g[base + 32:base + 48]
        g3 = g[base + 48:base + 64]
        poly = g0 + g1 * x + g2 * x2
        out_t = poly * jnp.exp(g3 * lnr) * fcut
        out_ref[...] = out_t[:_NING]


def kernel(r, bond_indices, edge_numbers, hopping_params, overlap_params,
           atomic_radius_list):
    f32 = jnp.float32
    B = r.shape[0]
    blk = _BLK
    nblk = B // blk
    nbt, ning, np_ = hopping_params.shape

    rad_pad = jnp.zeros((128,), f32).at[:atomic_radius_list.shape[0]].set(
        atomic_radius_list)
    r0 = _sc_r0(edge_numbers[0], edge_numbers[1], rad_pad)

    cols = []
    for tbl in (hopping_params, overlap_params):
        for k in range(np_):
            plane = tbl[:, :, k]
            if k == 3:
                plane = 1.0 + jnp.abs(plane)
            cols.append(jnp.pad(plane, ((0, 0), (0, 16 - ning))))
    wt = jnp.concatenate(cols, axis=1).T

    r3 = r.reshape(nblk, 1, blk)
    bt3 = bond_indices.reshape(nblk, 1, blk)
    r03 = r0.reshape(nblk, 1, blk)

    row_spec = pl.BlockSpec((1, 1, blk), lambda i: (i, 0, 0))
    out_spec = pl.BlockSpec((ning, blk), lambda i: (0, i))
    hop_t, ov_t = pl.pallas_call(
        _tc_body,
        grid=(nblk,),
        in_specs=[row_spec, row_spec, row_spec,
                  pl.BlockSpec((128, nbt), lambda i: (0, 0))],
        out_specs=[out_spec, out_spec],
        out_shape=[jax.ShapeDtypeStruct((ning, B), f32),
                   jax.ShapeDtypeStruct((ning, B), f32)],
    )(r3, bt3, r03, wt)
    return (hop_t.T, ov_t.T)


# speedup vs baseline: 7.4146x; 1.3246x over previous
import functools

import jax
import jax.numpy as jnp
from jax import lax
from jax.experimental import pallas as pl
from jax.experimental.pallas import tpu as pltpu
from jax.experimental.pallas import tpu_sc as plsc

_BLK = 8192
_NING = 13


def _r0_body(nw, b_per_w, nc, e0_hbm, e1_hbm, rad_hbm, out_hbm,
             e0_v, e1_v, rad_v, out_v):
    wid = lax.axis_index("s") * nc + lax.axis_index("c")
    base = wid * b_per_w
    pltpu.sync_copy(e0_hbm.at[pl.ds(base, b_per_w)], e0_v)
    pltpu.sync_copy(e1_hbm.at[pl.ds(base, b_per_w)], e1_v)
    pltpu.sync_copy(rad_hbm, rad_v)

    def step(k, _):
        sl = pl.ds(k * 16, 16)
        ia = e0_v[sl]
        ib = e1_v[sl]
        a = plsc.load_gather(rad_v, [ia])
        b = plsc.load_gather(rad_v, [ib])
        out_v[sl] = a + b
        return _

    lax.fori_loop(0, b_per_w // 16, step, None, unroll=4)
    pltpu.sync_copy(out_v, out_hbm.at[pl.ds(base, b_per_w)])


def _sc_r0(e0, e1, rad_pad):
    B = e0.shape[0]
    info = plsc.get_sparse_core_info()
    nc, ns = info.num_cores, info.num_subcores
    nw = nc * ns
    b_per_w = B // nw
    mesh = plsc.VectorSubcoreMesh(core_axis_name="c", subcore_axis_name="s")
    fn = functools.partial(
        pl.kernel,
        mesh=mesh,
        compiler_params=pltpu.CompilerParams(needs_layout_passes=False),
        out_type=jax.ShapeDtypeStruct((B,), jnp.float32),
        scratch_types=[
            pltpu.VMEM((b_per_w,), jnp.int32),
            pltpu.VMEM((b_per_w,), jnp.int32),
            pltpu.VMEM((128,), jnp.float32),
            pltpu.VMEM((b_per_w,), jnp.float32),
        ],
    )(functools.partial(_r0_body, nw, b_per_w, nc))
    return fn(e0, e1, rad_pad)


def _tc_body(r_ref, bt_ref, r0_ref, wt_ref, hop_ref, ov_ref):
    f32 = jnp.float32
    rrow = r_ref[0]
    bt = bt_ref[0]
    r0 = r0_ref[0]
    nbt = wt_ref.shape[1]
    blk = rrow.shape[1]

    iota_t = lax.broadcasted_iota(jnp.int32, (nbt, blk), 0)
    oht = jnp.where(iota_t == bt, 1.0, 0.0).astype(f32)

    g = lax.dot_general(wt_ref[...], oht, (((1,), (0,)), ((), ())),
                        preferred_element_type=f32)

    x = rrow / r0 - 1.0
    x2 = x * x
    lnr = jnp.log(r0 / rrow)
    fcut = 1.0 / (1.0 + jnp.exp((rrow - 5.0) * 5.0))

    for t, out_ref in ((0, hop_ref), (1, ov_ref)):
        base = 64 * t
        g0 = g[base:base + 16]
        g1 = g[base + 16:base + 32]
        g2 = g[base + 32:base + 48]
        g3 = g[base + 48:base + 64]
        poly = g0 + g1 * x + g2 * x2
        out_t = poly * jnp.exp(g3 * lnr) * fcut
        out_ref[...] = out_t[:_NING]


def kernel(r, bond_indices, edge_numbers, hopping_params, overlap_params,
           atomic_radius_list):
    f32 = jnp.float32
    B = r.shape[0]
    blk = _BLK
    nblk = B // blk
    nbt, ning, np_ = hopping_params.shape

    rad_pad = jnp.zeros((128,), f32).at[:atomic_radius_list.shape[0]].set(
        atomic_radius_list)
    r0 = _sc_r0(edge_numbers[0], edge_numbers[1], rad_pad)

    cols = []
    for tbl in (hopping_params, overlap_params):
        for k in range(np_):
            plane = tbl[:, :, k]
            if k == 3:
                plane = 1.0 + jnp.abs(plane)
            cols.append(jnp.pad(plane, ((0, 0), (0, 16 - ning))))
    wt = jnp.concatenate(cols, axis=1).T

    r3 = r.reshape(nblk, 1, blk)
    bt3 = bond_indices.reshape(nblk, 1, blk)
    r03 = r0.reshape(nblk, 1, blk)

    row_spec = pl.BlockSpec((1, 1, blk), lambda i: (i, 0, 0))
    out_spec = pl.BlockSpec((ning, blk), lambda i: (0, i))
    hop_t, ov_t = pl.pallas_call(
        _tc_body,
        grid=(nblk,),
        in_specs=[row_spec, row_spec, row_spec,
                  pl.BlockSpec((128, nbt), lambda i: (0, 0))],
        out_specs=[out_spec, out_spec],
        out_shape=[jax.ShapeDtypeStruct((ning, B), f32),
                   jax.ShapeDtypeStruct((ning, B), f32)],
    )(r3, bt3, r03, wt)
    return (hop_t.T, ov_t.T)


# speedup vs baseline: 8.8780x; 1.1974x over previous
import functools

import jax
import jax.numpy as jnp
from jax import lax
from jax.experimental import pallas as pl
from jax.experimental.pallas import tpu as pltpu
from jax.experimental.pallas import tpu_sc as plsc

_BLK = 16384
_NING = 13


def _r0_body(nw, b_per_w, nc, e0_hbm, e1_hbm, rad_hbm, out_hbm,
             e0_v, e1_v, rad_v, out_v):
    wid = lax.axis_index("s") * nc + lax.axis_index("c")
    base = wid * b_per_w
    pltpu.sync_copy(e0_hbm.at[pl.ds(base, b_per_w)], e0_v)
    pltpu.sync_copy(e1_hbm.at[pl.ds(base, b_per_w)], e1_v)
    pltpu.sync_copy(rad_hbm, rad_v)

    def step(k, _):
        sl = pl.ds(k * 16, 16)
        ia = e0_v[sl]
        ib = e1_v[sl]
        a = plsc.load_gather(rad_v, [ia])
        b = plsc.load_gather(rad_v, [ib])
        out_v[sl] = a + b
        return _

    lax.fori_loop(0, b_per_w // 16, step, None, unroll=4)
    pltpu.sync_copy(out_v, out_hbm.at[pl.ds(base, b_per_w)])


def _sc_r0(e0, e1, rad_pad):
    B = e0.shape[0]
    info = plsc.get_sparse_core_info()
    nc, ns = info.num_cores, info.num_subcores
    nw = nc * ns
    b_per_w = B // nw
    mesh = plsc.VectorSubcoreMesh(core_axis_name="c", subcore_axis_name="s")
    fn = functools.partial(
        pl.kernel,
        mesh=mesh,
        compiler_params=pltpu.CompilerParams(needs_layout_passes=False),
        out_type=jax.ShapeDtypeStruct((B,), jnp.float32),
        scratch_types=[
            pltpu.VMEM((b_per_w,), jnp.int32),
            pltpu.VMEM((b_per_w,), jnp.int32),
            pltpu.VMEM((128,), jnp.float32),
            pltpu.VMEM((b_per_w,), jnp.float32),
        ],
    )(functools.partial(_r0_body, nw, b_per_w, nc))
    return fn(e0, e1, rad_pad)


def _tc_body(r_ref, bt_ref, r0_ref, wt_ref, hop_ref, ov_ref):
    f32 = jnp.float32
    rrow = r_ref[0]
    bt = bt_ref[0]
    r0 = r0_ref[0]
    nbt = wt_ref.shape[1]
    blk = rrow.shape[1]

    iota_t = lax.broadcasted_iota(jnp.int32, (nbt, blk), 0)
    oht = jnp.where(iota_t == bt, 1.0, 0.0).astype(f32)

    g = lax.dot_general(wt_ref[...], oht, (((1,), (0,)), ((), ())),
                        preferred_element_type=f32)

    x = rrow / r0 - 1.0
    x2 = x * x
    lnr = jnp.log(r0 / rrow)
    fcut = 1.0 / (1.0 + jnp.exp((rrow - 5.0) * 5.0))

    for t, out_ref in ((0, hop_ref), (1, ov_ref)):
        base = 64 * t
        g0 = g[base:base + 16]
        g1 = g[base + 16:base + 32]
        g2 = g[base + 32:base + 48]
        g3 = g[base + 48:base + 64]
        poly = g0 + g1 * x + g2 * x2
        out_t = poly * jnp.exp(g3 * lnr) * fcut
        out_ref[...] = out_t[:_NING]


def kernel(r, bond_indices, edge_numbers, hopping_params, overlap_params,
           atomic_radius_list):
    f32 = jnp.float32
    B = r.shape[0]
    blk = _BLK
    nblk = B // blk
    nbt, ning, np_ = hopping_params.shape

    rad_pad = jnp.zeros((128,), f32).at[:atomic_radius_list.shape[0]].set(
        atomic_radius_list)
    r0 = _sc_r0(edge_numbers[0], edge_numbers[1], rad_pad)

    cols = []
    for tbl in (hopping_params, overlap_params):
        for k in range(np_):
            plane = tbl[:, :, k]
            if k == 3:
                plane = 1.0 + jnp.abs(plane)
            cols.append(jnp.pad(plane, ((0, 0), (0, 16 - ning))))
    wt = jnp.concatenate(cols, axis=1).T

    r3 = r.reshape(nblk, 1, blk)
    bt3 = bond_indices.reshape(nblk, 1, blk)
    r03 = r0.reshape(nblk, 1, blk)

    row_spec = pl.BlockSpec((1, 1, blk), lambda i: (i, 0, 0))
    out_spec = pl.BlockSpec((ning, blk), lambda i: (0, i))
    hop_t, ov_t = pl.pallas_call(
        _tc_body,
        grid=(nblk,),
        in_specs=[row_spec, row_spec, row_spec,
                  pl.BlockSpec((128, nbt), lambda i: (0, 0))],
        out_specs=[out_spec, out_spec],
        out_shape=[jax.ShapeDtypeStruct((ning, B), f32),
                   jax.ShapeDtypeStruct((ning, B), f32)],
    )(r3, bt3, r03, wt)
    return (hop_t.T, ov_t.T)


# speedup vs baseline: 9.0651x; 1.0211x over previous
import functools

import jax
import jax.numpy as jnp
from jax import lax
from jax.experimental import pallas as pl
from jax.experimental.pallas import tpu as pltpu
from jax.experimental.pallas import tpu_sc as plsc

_BLK = 32768
_NING = 13


def _r0_body(nw, b_per_w, nc, e0_hbm, e1_hbm, rad_hbm, out_hbm,
             e0_v, e1_v, rad_v, out_v):
    wid = lax.axis_index("s") * nc + lax.axis_index("c")
    base = wid * b_per_w
    pltpu.sync_copy(e0_hbm.at[pl.ds(base, b_per_w)], e0_v)
    pltpu.sync_copy(e1_hbm.at[pl.ds(base, b_per_w)], e1_v)
    pltpu.sync_copy(rad_hbm, rad_v)

    def step(k, _):
        sl = pl.ds(k * 16, 16)
        ia = e0_v[sl]
        ib = e1_v[sl]
        a = plsc.load_gather(rad_v, [ia])
        b = plsc.load_gather(rad_v, [ib])
        out_v[sl] = a + b
        return _

    lax.fori_loop(0, b_per_w // 16, step, None, unroll=4)
    pltpu.sync_copy(out_v, out_hbm.at[pl.ds(base, b_per_w)])


def _sc_r0(e0, e1, rad_pad):
    B = e0.shape[0]
    info = plsc.get_sparse_core_info()
    nc, ns = info.num_cores, info.num_subcores
    nw = nc * ns
    b_per_w = B // nw
    mesh = plsc.VectorSubcoreMesh(core_axis_name="c", subcore_axis_name="s")
    fn = functools.partial(
        pl.kernel,
        mesh=mesh,
        compiler_params=pltpu.CompilerParams(needs_layout_passes=False),
        out_type=jax.ShapeDtypeStruct((B,), jnp.float32),
        scratch_types=[
            pltpu.VMEM((b_per_w,), jnp.int32),
            pltpu.VMEM((b_per_w,), jnp.int32),
            pltpu.VMEM((128,), jnp.float32),
            pltpu.VMEM((b_per_w,), jnp.float32),
        ],
    )(functools.partial(_r0_body, nw, b_per_w, nc))
    return fn(e0, e1, rad_pad)


def _tc_body(r_ref, bt_ref, r0_ref, wt_ref, hop_ref, ov_ref):
    f32 = jnp.float32
    rrow = r_ref[0]
    bt = bt_ref[0]
    r0 = r0_ref[0]
    nbt = wt_ref.shape[1]
    blk = rrow.shape[1]

    iota_t = lax.broadcasted_iota(jnp.int32, (nbt, blk), 0)
    oht = jnp.where(iota_t == bt, 1.0, 0.0).astype(f32)

    g = lax.dot_general(wt_ref[...], oht, (((1,), (0,)), ((), ())),
                        preferred_element_type=f32)

    x = rrow / r0 - 1.0
    x2 = x * x
    lnr = jnp.log(r0 / rrow)
    fcut = 1.0 / (1.0 + jnp.exp((rrow - 5.0) * 5.0))

    for t, out_ref in ((0, hop_ref), (1, ov_ref)):
        base = 64 * t
        g0 = g[base:base + 16]
        g1 = g[base + 16:base + 32]
        g2 = g[base + 32:base + 48]
        g3 = g[base + 48:base + 64]
        poly = g0 + g1 * x + g2 * x2
        out_t = poly * jnp.exp(g3 * lnr) * fcut
        out_ref[...] = out_t[:_NING]


def kernel(r, bond_indices, edge_numbers, hopping_params, overlap_params,
           atomic_radius_list):
    f32 = jnp.float32
    B = r.shape[0]
    blk = _BLK
    nblk = B // blk
    nbt, ning, np_ = hopping_params.shape

    rad_pad = jnp.zeros((128,), f32).at[:atomic_radius_list.shape[0]].set(
        atomic_radius_list)
    r0 = _sc_r0(edge_numbers[0], edge_numbers[1], rad_pad)

    cols = []
    for tbl in (hopping_params, overlap_params):
        for k in range(np_):
            plane = tbl[:, :, k]
            if k == 3:
                plane = 1.0 + jnp.abs(plane)
            cols.append(jnp.pad(plane, ((0, 0), (0, 16 - ning))))
    wt = jnp.concatenate(cols, axis=1).T

    r3 = r.reshape(nblk, 1, blk)
    bt3 = bond_indices.reshape(nblk, 1, blk)
    r03 = r0.reshape(nblk, 1, blk)

    row_spec = pl.BlockSpec((1, 1, blk), lambda i: (i, 0, 0))
    out_spec = pl.BlockSpec((ning, blk), lambda i: (0, i))
    hop_t, ov_t = pl.pallas_call(
        _tc_body,
        grid=(nblk,),
        in_specs=[row_spec, row_spec, row_spec,
                  pl.BlockSpec((128, nbt), lambda i: (0, 0))],
        out_specs=[out_spec, out_spec],
        out_shape=[jax.ShapeDtypeStruct((ning, B), f32),
                   jax.ShapeDtypeStruct((ning, B), f32)],
    )(r3, bt3, r03, wt)
    return (hop_t.T, ov_t.T)


# speedup vs baseline: 9.8081x; 1.0820x over previous
import functools

import jax
import jax.numpy as jnp
from jax import lax
from jax.experimental import pallas as pl
from jax.experimental.pallas import tpu as pltpu
from jax.experimental.pallas import tpu_sc as plsc

_BLK = 32768
_NING = 13


def _r0_body(nw, b_per_w, nc, e_hbm, rad_hbm, out_hbm,
             e_v, rad_v, out_v):
    wid = lax.axis_index("s") * nc + lax.axis_index("c")
    base = wid * b_per_w
    pltpu.sync_copy(e_hbm.at[:, pl.ds(base, b_per_w)], e_v)
    pltpu.sync_copy(rad_hbm, rad_v)

    def step(k, _):
        sl = pl.ds(k * 16, 16)
        ia = e_v[0, sl]
        ib = e_v[1, sl]
        a = plsc.load_gather(rad_v, [ia])
        b = plsc.load_gather(rad_v, [ib])
        out_v[sl] = a + b
        return _

    lax.fori_loop(0, b_per_w // 16, step, None, unroll=4)
    pltpu.sync_copy(out_v, out_hbm.at[pl.ds(base, b_per_w)])


def _sc_r0(edge_numbers, rad_pad):
    B = edge_numbers.shape[1]
    info = plsc.get_sparse_core_info()
    nc, ns = info.num_cores, info.num_subcores
    nw = nc * ns
    b_per_w = B // nw
    mesh = plsc.VectorSubcoreMesh(core_axis_name="c", subcore_axis_name="s")
    fn = functools.partial(
        pl.kernel,
        mesh=mesh,
        compiler_params=pltpu.CompilerParams(needs_layout_passes=False),
        out_type=jax.ShapeDtypeStruct((B,), jnp.float32),
        scratch_types=[
            pltpu.VMEM((2, b_per_w), jnp.int32),
            pltpu.VMEM((128,), jnp.float32),
            pltpu.VMEM((b_per_w,), jnp.float32),
        ],
    )(functools.partial(_r0_body, nw, b_per_w, nc))
    return fn(edge_numbers, rad_pad)


def _tc_body(r_ref, bt_ref, r0_ref, wt_ref, hop_ref, ov_ref):
    f32 = jnp.float32
    rrow = r_ref[0]
    bt = bt_ref[0]
    r0 = r0_ref[0]
    nbt = wt_ref.shape[1]
    blk = rrow.shape[1]

    iota_t = lax.broadcasted_iota(jnp.int32, (nbt, blk), 0)
    oht = jnp.where(iota_t == bt, 1.0, 0.0).astype(f32)

    g = lax.dot_general(wt_ref[...], oht, (((1,), (0,)), ((), ())),
                        preferred_element_type=f32)

    x = rrow / r0 - 1.0
    x2 = x * x
    lnr = jnp.log(r0 / rrow)
    fcut = 1.0 / (1.0 + jnp.exp((rrow - 5.0) * 5.0))

    for t, out_ref in ((0, hop_ref), (1, ov_ref)):
        base = 64 * t
        g0 = g[base:base + 16]
        g1 = g[base + 16:base + 32]
        g2 = g[base + 32:base + 48]
        g3 = g[base + 48:base + 64]
        poly = g0 + g1 * x + g2 * x2
        out_t = poly * jnp.exp(g3 * lnr) * fcut
        out_ref[...] = out_t[:_NING]


def kernel(r, bond_indices, edge_numbers, hopping_params, overlap_params,
           atomic_radius_list):
    f32 = jnp.float32
    B = r.shape[0]
    blk = _BLK
    nblk = B // blk
    nbt, ning, np_ = hopping_params.shape

    rad_pad = jnp.zeros((128,), f32).at[:atomic_radius_list.shape[0]].set(
        atomic_radius_list)
    r0 = _sc_r0(edge_numbers, rad_pad)

    cols = []
    for tbl in (hopping_params, overlap_params):
        for k in range(np_):
            plane = tbl[:, :, k]
            if k == 3:
                plane = 1.0 + jnp.abs(plane)
            cols.append(jnp.pad(plane, ((0, 0), (0, 16 - ning))))
    wt = jnp.concatenate(cols, axis=1).T

    r3 = r.reshape(nblk, 1, blk)
    bt3 = bond_indices.reshape(nblk, 1, blk)
    r03 = r0.reshape(nblk, 1, blk)

    row_spec = pl.BlockSpec((1, 1, blk), lambda i: (i, 0, 0))
    out_spec = pl.BlockSpec((ning, blk), lambda i: (0, i))
    hop_t, ov_t = pl.pallas_call(
        _tc_body,
        grid=(nblk,),
        in_specs=[row_spec, row_spec, row_spec,
                  pl.BlockSpec((128, nbt), lambda i: (0, 0))],
        out_specs=[out_spec, out_spec],
        out_shape=[jax.ShapeDtypeStruct((ning, B), f32),
                   jax.ShapeDtypeStruct((ning, B), f32)],
    )(r3, bt3, r03, wt)
    return (hop_t.T, ov_t.T)
